# baseline (speedup 1.0000x reference)
import jax
import jax.numpy as jnp
from jax.experimental import pallas as pl
from jax.experimental.pallas import tpu as pltpu
from functools import partial
import numpy as np

B, N, D, H, OUT = 512, 16, 128, 128, 32
STEPS = 8
BS = B // STEPS
CH = 8
Gs = BS // CH
Rs = Gs * N


def _gcn_block(x_ref, a_ref, bdm_ref, scm_ref, e_ref, w1_ref, b1_ref,
               w2_ref, b2_ref, w3_ref, b3_ref, out_ref):
    f32 = jnp.float32
    mm = partial(jnp.dot, preferred_element_type=f32)
    w1 = w1_ref[...]
    w2 = w2_ref[...]
    w3 = w3_ref[...]
    e = e_ref[...]
    bdm = bdm_ref[...]
    scm = scm_ref[...]
    cseg = [slice(c * Gs, (c + 1) * Gs) for c in range(CH)]

    ar = [a_ref[cseg[c]].reshape(Rs, N) for c in range(CH)]
    BD = [mm(ar[c], e) * bdm for c in range(CH)]

    p1 = [mm(x_ref[cseg[c]].reshape(Rs, D), w1) for c in range(CH)]
    h1 = [jax.nn.relu(mm(BD[c], p1[c]) + b1_ref[...]) for c in range(CH)]
    p2 = [mm(h1[c], w2) for c in range(CH)]
    h2 = [jax.nn.relu(mm(BD[c], p2[c]) + b2_ref[...]) for c in range(CH)]

    cs = [mm(scm, ar[c]) * (1.0 / N) for c in range(CH)]
    Sc = [mm(cs[c], e) * scm for c in range(CH)]
    z = [mm(Sc[c], h2[c]) for c in range(CH)]
    for c in range(CH):
        out_ref[cseg[c], :] = mm(z[c], w3) + b3_ref[...]


def _masks():
    rg = np.arange(Rs)[:, None] // N
    cg = np.arange(Rs)[None, :] // N
    bdm = (rg == cg).astype(np.float32)
    og = np.arange(Gs)[:, None]
    scm = (og == cg.reshape(1, Rs)).astype(np.float32)
    e = (np.arange(N)[:, None] == np.arange(Rs)[None, :] % N)
    return jnp.asarray(bdm), jnp.asarray(scm), jnp.asarray(e, dtype=np.float32)


def kernel(batch_graph, adj, W1, b1, W2, b2, W3, b3):
    bdm, scm, e = _masks()
    out = pl.pallas_call(
        _gcn_block,
        grid=(STEPS,),
        in_specs=[
            pl.BlockSpec((BS, N, D), lambda i: (i, 0, 0)),
            pl.BlockSpec((BS, N, N), lambda i: (i, 0, 0)),
            pl.BlockSpec((Rs, Rs), lambda i: (0, 0)),
            pl.BlockSpec((Gs, Rs), lambda i: (0, 0)),
            pl.BlockSpec((N, Rs), lambda i: (0, 0)),
            pl.BlockSpec((D, H), lambda i: (0, 0)),
            pl.BlockSpec((1, H), lambda i: (0, 0)),
            pl.BlockSpec((H, H // 2), lambda i: (0, 0)),
            pl.BlockSpec((1, H // 2), lambda i: (0, 0)),
            pl.BlockSpec((H // 2, OUT), lambda i: (0, 0)),
            pl.BlockSpec((1, OUT), lambda i: (0, 0)),
        ],
        out_specs=pl.BlockSpec((BS, OUT), lambda i: (i, 0)),
        out_shape=jax.ShapeDtypeStruct((B, OUT), jnp.float32),
    )(batch_graph, adj, bdm, scm, e, W1, b1.reshape(1, H), W2,
      b2.reshape(1, H // 2), W3, b3.reshape(1, OUT))
    return out.reshape(B, OUT, 1, 1)


# speedup vs baseline: 1.2240x; 1.2240x over previous
import jax
import jax.numpy as jnp
from jax.experimental import pallas as pl
from jax.experimental.pallas import tpu as pltpu
from functools import partial
import numpy as np

B, N, D, H, OUT = 512, 16, 128, 128, 32
STEPS = 2
BS = B // STEPS
CH = 32
Gs = BS // CH
Rs = Gs * N


def _gcn_block(x_ref, a_ref, bdm_ref, scm_ref, e_ref, w1_ref, b1_ref,
               w2_ref, b2_ref, w3_ref, b3_ref, out_ref):
    f32 = jnp.float32
    mm = partial(jnp.dot, preferred_element_type=f32)
    w1 = w1_ref[...]
    w2 = w2_ref[...]
    w3 = w3_ref[...]
    e = e_ref[...]
    bdm = bdm_ref[...]
    scm = scm_ref[...]
    cseg = [slice(c * Gs, (c + 1) * Gs) for c in range(CH)]

    ar = [a_ref[cseg[c]].reshape(Rs, N) for c in range(CH)]
    BD = [mm(ar[c], e) * bdm for c in range(CH)]

    p1 = [mm(x_ref[cseg[c]].reshape(Rs, D), w1) for c in range(CH)]
    h1 = [jax.nn.relu(mm(BD[c], p1[c]) + b1_ref[...]) for c in range(CH)]
    p2 = [mm(h1[c], w2) for c in range(CH)]
    h2 = [jax.nn.relu(mm(BD[c], p2[c]) + b2_ref[...]) for c in range(CH)]

    cs = [mm(scm, ar[c]) * (1.0 / N) for c in range(CH)]
    Sc = [mm(cs[c], e) * scm for c in range(CH)]
    z = [mm(Sc[c], h2[c]) for c in range(CH)]
    for c in range(CH):
        out_ref[cseg[c], :] = mm(z[c], w3) + b3_ref[...]


def _masks():
    rg = np.arange(Rs)[:, None] // N
    cg = np.arange(Rs)[None, :] // N
    bdm = (rg == cg).astype(np.float32)
    og = np.arange(Gs)[:, None]
    scm = (og == cg.reshape(1, Rs)).astype(np.float32)
    e = (np.arange(N)[:, None] == np.arange(Rs)[None, :] % N)
    return jnp.asarray(bdm), jnp.asarray(scm), jnp.asarray(e, dtype=np.float32)


def kernel(batch_graph, adj, W1, b1, W2, b2, W3, b3):
    bdm, scm, e = _masks()
    out = pl.pallas_call(
        _gcn_block,
        grid=(STEPS,),
        in_specs=[
            pl.BlockSpec((BS, N, D), lambda i: (i, 0, 0)),
            pl.BlockSpec((BS, N, N), lambda i: (i, 0, 0)),
            pl.BlockSpec((Rs, Rs), lambda i: (0, 0)),
            pl.BlockSpec((Gs, Rs), lambda i: (0, 0)),
            pl.BlockSpec((N, Rs), lambda i: (0, 0)),
            pl.BlockSpec((D, H), lambda i: (0, 0)),
            pl.BlockSpec((1, H), lambda i: (0, 0)),
            pl.BlockSpec((H, H // 2), lambda i: (0, 0)),
            pl.BlockSpec((1, H // 2), lambda i: (0, 0)),
            pl.BlockSpec((H // 2, OUT), lambda i: (0, 0)),
            pl.BlockSpec((1, OUT), lambda i: (0, 0)),
        ],
        out_specs=pl.BlockSpec((BS, OUT), lambda i: (i, 0)),
        out_shape=jax.ShapeDtypeStruct((B, OUT), jnp.float32),
    )(batch_graph, adj, bdm, scm, e, W1, b1.reshape(1, H), W2,
      b2.reshape(1, H // 2), W3, b3.reshape(1, OUT))
    return out.reshape(B, OUT, 1, 1)
